# Initial kernel scaffold; baseline (speedup 1.0000x reference)
import jax
import jax.numpy as jnp
from jax.experimental import pallas as pl


def kernel(rtg, state, action, pos_table):
    raise NotImplementedError("write your pallas kernel here")


# speedup vs baseline: 1.5224x; 1.5224x over previous
import jax
import jax.numpy as jnp
from jax.experimental import pallas as pl


def _body(rtg_ref, state_ref, action_ref, pos_ref, o0, o1, o2):
    p = pos_ref[...]
    o0[0] = rtg_ref[0] + p
    o1[0] = state_ref[0] + p
    o2[0] = action_ref[0] + p


def kernel(rtg, state, action, pos_table):
    B, S, D = rtg.shape
    BS = 512
    x_spec = pl.BlockSpec((1, BS, D), lambda s, b: (b, s, 0))
    pos_spec = pl.BlockSpec((BS, D), lambda s, b: (s, 0))
    out_shape = jax.ShapeDtypeStruct((B, S, D), rtg.dtype)
    return pl.pallas_call(
        _body,
        grid=(S // BS, B),
        in_specs=[x_spec, x_spec, x_spec, pos_spec],
        out_specs=[x_spec, x_spec, x_spec],
        out_shape=[out_shape, out_shape, out_shape],
    )(rtg, state, action, pos_table[:S])
